# Initial kernel scaffold; baseline (speedup 1.0000x reference)
import functools

import jax
import jax.numpy as jnp
from jax import lax
from jax.experimental import pallas as pl
from jax.experimental.pallas import tpu as pltpu
from jax.experimental.pallas import tpu_sc as plsc

T, D, E, DFF = 16384, 768, 64, 768
CAP = 384
NR = E * CAP + CAP
TRASH = E * CAP
BT = 1024
NB = T // BT
NW = 32
TPW = T // NW
CH = 128
NCH = TPW // CH


def _router_body(x_ref, wg_ref, slot_ref, gate_ref, counts_ref):
    i = pl.program_id(0)

    @pl.when(i == 0)
    def _init():
        counts_ref[...] = jnp.zeros_like(counts_ref)

    xblk = x_ref[...]
    logits = jnp.dot(xblk, wg_ref[...],
                     preferred_element_type=jnp.float32)
    m = jnp.max(logits, axis=1, keepdims=True)
    denom = jnp.sum(jnp.exp(logits - m), axis=1, keepdims=True)
    gate = 1.0 / denom
    eiota = lax.broadcasted_iota(jnp.int32, (BT, E), 1)
    expert = jnp.min(jnp.where(logits == m, eiota, E),
                     axis=1, keepdims=True)
    onehot = (eiota == expert).astype(jnp.float32)
    r_io = lax.broadcasted_iota(jnp.int32, (BT, BT), 0)
    c_io = lax.broadcasted_iota(jnp.int32, (BT, BT), 1)
    tri = (c_io <= r_io).astype(jnp.float32)
    incl = jnp.dot(tri, onehot, preferred_element_type=jnp.float32)
    prev = counts_ref[0:1, :]
    pos = jnp.sum((incl - 1.0 + prev) * onehot, axis=1,
                  keepdims=True).astype(jnp.int32)
    slot = jnp.where(pos < CAP, expert * CAP + pos, TRASH)
    slot_ref[...] = slot
    gate_ref[...] = jnp.broadcast_to(gate, (BT, 16))
    counts_ref[0:1, :] = prev + jnp.sum(onehot, axis=0, keepdims=True)


_router = pl.pallas_call(
    _router_body,
    grid=(NB,),
    in_specs=[
        pl.BlockSpec((BT, D), lambda i: (i, 0)),
        pl.BlockSpec((D, E), lambda i: (0, 0)),
    ],
    out_specs=[
        pl.BlockSpec((BT, 1), lambda i: (i, 0)),
        pl.BlockSpec((BT, 16), lambda i: (i, 0)),
    ],
    out_shape=[
        jax.ShapeDtypeStruct((T, 1), jnp.int32),
        jax.ShapeDtypeStruct((T, 16), jnp.float32),
    ],
    scratch_shapes=[pltpu.VMEM((8, E), jnp.float32)],
)


def _ffn_body(buf_ref, g_ref, w1_ref, b1_ref, w2_ref, b2_ref, y_ref):
    e = pl.program_id(0)

    @pl.when(e < E)
    def _compute():
        a = buf_ref[...]
        h = jnp.dot(a, w1_ref[0], preferred_element_type=jnp.float32)
        h = jnp.maximum(h + b1_ref[...], 0.0)
        y = jnp.dot(h, w2_ref[0], preferred_element_type=jnp.float32)
        y = y + b2_ref[...]
        y_ref[...] = y * g_ref[:, 0:1]

    @pl.when(e == E)
    def _zeros():
        y_ref[...] = jnp.zeros_like(y_ref)


_ffn = pl.pallas_call(
    _ffn_body,
    grid=(E + 1,),
    in_specs=[
        pl.BlockSpec((CAP, D), lambda e: (jnp.minimum(e, E - 1), 0)),
        pl.BlockSpec((CAP, 16), lambda e: (jnp.minimum(e, E - 1), 0)),
        pl.BlockSpec((1, D, DFF), lambda e: (jnp.minimum(e, E - 1), 0, 0)),
        pl.BlockSpec((1, DFF), lambda e: (jnp.minimum(e, E - 1), 0)),
        pl.BlockSpec((1, DFF, D), lambda e: (jnp.minimum(e, E - 1), 0, 0)),
        pl.BlockSpec((1, D), lambda e: (jnp.minimum(e, E - 1), 0)),
    ],
    out_specs=pl.BlockSpec((CAP, D), lambda e: (e, 0)),
    out_shape=jax.ShapeDtypeStruct((NR, D), jnp.float32),
)


_sc_mesh = plsc.VectorSubcoreMesh(core_axis_name="c", subcore_axis_name="s")


@functools.partial(
    pl.kernel,
    mesh=_sc_mesh,
    out_type=[
        jax.ShapeDtypeStruct((NR, D), jnp.float32),
        jax.ShapeDtypeStruct((NR, 16), jnp.float32),
    ],
    scratch_types=[
        pltpu.VMEM((NCH, CH), jnp.int32),
        pltpu.VMEM((CH, D), jnp.float32),
        pltpu.VMEM((CH, 16), jnp.float32),
        pltpu.SemaphoreType.DMA,
        pltpu.SemaphoreType.DMA,
    ],
)
def _scatter(slot_hbm, x_hbm, aux_hbm, buf_hbm, grow_hbm,
             idx_v, row_v, aux_v, sem1, sem2):
    wid = lax.axis_index("s") * 2 + lax.axis_index("c")
    base = wid * TPW
    for j in range(NCH):
        pltpu.sync_copy(slot_hbm.at[pl.ds(base + j * CH, CH)], idx_v.at[j])
    for j in range(NCH):
        pltpu.sync_copy(x_hbm.at[pl.ds(base + j * CH, CH)], row_v)
        pltpu.async_copy(row_v, buf_hbm.at[idx_v.at[j]], sem1).wait()
        pltpu.sync_copy(aux_hbm.at[pl.ds(base + j * CH, CH)], aux_v)
        pltpu.async_copy(aux_v, grow_hbm.at[idx_v.at[j]], sem2).wait()


@functools.partial(
    pl.kernel,
    mesh=_sc_mesh,
    out_type=jax.ShapeDtypeStruct((T, D), jnp.float32),
    scratch_types=[
        pltpu.VMEM((NCH, CH), jnp.int32),
        pltpu.VMEM((CH, D), jnp.float32),
        pltpu.SemaphoreType.DMA,
    ],
)
def _gather(slot_hbm, y_hbm, out_hbm, idx_v, row_v, sem):
    wid = lax.axis_index("s") * 2 + lax.axis_index("c")
    base = wid * TPW
    for j in range(NCH):
        pltpu.sync_copy(slot_hbm.at[pl.ds(base + j * CH, CH)], idx_v.at[j])
    for j in range(NCH):
        pltpu.async_copy(y_hbm.at[idx_v.at[j]], row_v, sem).wait()
        pltpu.sync_copy(row_v, out_hbm.at[pl.ds(base + j * CH, CH)])


def kernel(x, Wg, W1, b1, W2, b2):
    slot2d, aux = _router(x, Wg)
    slot = slot2d.reshape(T)
    buf, grow = _scatter(slot, x, aux)
    y = _ffn(buf, grow, W1, b1, W2, b2)
    return _gather(slot, y)


# speedup vs baseline: 2.9523x; 2.9523x over previous
import functools

import jax
import jax.numpy as jnp
from jax import lax
from jax.experimental import pallas as pl
from jax.experimental.pallas import tpu as pltpu
from jax.experimental.pallas import tpu_sc as plsc

T, D, E, DFF = 16384, 768, 64, 768
CAP = 384
NR = E * CAP + CAP
TRASH = E * CAP
BT = 1024
NB = T // BT
NW = 32
TPW = T // NW
CH = 128
NCH = TPW // CH
GW = 128


def _router_body(x_ref, wg_ref, slot_ref, gate_ref, counts_ref):
    i = pl.program_id(0)

    @pl.when(i == 0)
    def _init():
        counts_ref[...] = jnp.zeros_like(counts_ref)

    xblk = x_ref[...]
    logits = jnp.dot(xblk, wg_ref[...],
                     preferred_element_type=jnp.float32)
    m = jnp.max(logits, axis=1, keepdims=True)
    denom = jnp.sum(jnp.exp(logits - m), axis=1, keepdims=True)
    gate = 1.0 / denom
    eiota = lax.broadcasted_iota(jnp.int32, (BT, E), 1)
    expert = jnp.min(jnp.where(logits == m, eiota, E),
                     axis=1, keepdims=True)
    onehot = (eiota == expert).astype(jnp.float32)
    r_io = lax.broadcasted_iota(jnp.int32, (BT, BT), 0)
    c_io = lax.broadcasted_iota(jnp.int32, (BT, BT), 1)
    tri = (c_io <= r_io).astype(jnp.float32)
    incl = jnp.dot(tri, onehot, preferred_element_type=jnp.float32)
    prev = counts_ref[0:1, :]
    pos = jnp.sum((incl - 1.0 + prev) * onehot, axis=1,
                  keepdims=True).astype(jnp.int32)
    slot = jnp.where(pos < CAP, expert * CAP + pos, TRASH)
    slot_ref[...] = slot
    gate_ref[...] = jnp.broadcast_to(gate, (BT, GW))
    counts_ref[0:1, :] = prev + jnp.sum(onehot, axis=0, keepdims=True)


_router = pl.pallas_call(
    _router_body,
    grid=(NB,),
    in_specs=[
        pl.BlockSpec((BT, D), lambda i: (i, 0)),
        pl.BlockSpec((D, E), lambda i: (0, 0)),
    ],
    out_specs=[
        pl.BlockSpec((BT, 1), lambda i: (i, 0)),
        pl.BlockSpec((BT, GW), lambda i: (i, 0)),
    ],
    out_shape=[
        jax.ShapeDtypeStruct((T, 1), jnp.int32),
        jax.ShapeDtypeStruct((T, GW), jnp.float32),
    ],
    scratch_shapes=[pltpu.VMEM((8, E), jnp.float32)],
)


def _ffn_body(buf_ref, g_ref, w1_ref, b1_ref, w2_ref, b2_ref, y_ref):
    e = pl.program_id(0)

    @pl.when(e < E)
    def _compute():
        emin = jnp.minimum(e, E - 1)
        a = buf_ref[...]
        h = jnp.dot(a, w1_ref[0], preferred_element_type=jnp.float32)
        h = jnp.maximum(h + b1_ref[pl.ds(emin, 1), :], 0.0)
        y = jnp.dot(h, w2_ref[0], preferred_element_type=jnp.float32)
        y = y + b2_ref[pl.ds(emin, 1), :]
        y_ref[...] = y * g_ref[:, 0:1]

    @pl.when(e == E)
    def _zeros():
        y_ref[...] = jnp.zeros_like(y_ref)


_ffn = pl.pallas_call(
    _ffn_body,
    grid=(E + 1,),
    in_specs=[
        pl.BlockSpec((CAP, D), lambda e: (jnp.minimum(e, E - 1), 0)),
        pl.BlockSpec((CAP, GW), lambda e: (jnp.minimum(e, E - 1), 0)),
        pl.BlockSpec((1, D, DFF), lambda e: (jnp.minimum(e, E - 1), 0, 0)),
        pl.BlockSpec((E, DFF), lambda e: (0, 0)),
        pl.BlockSpec((1, DFF, D), lambda e: (jnp.minimum(e, E - 1), 0, 0)),
        pl.BlockSpec((E, D), lambda e: (0, 0)),
    ],
    out_specs=pl.BlockSpec((CAP, D), lambda e: (e, 0)),
    out_shape=jax.ShapeDtypeStruct((NR, D), jnp.float32),
)


@functools.cache
def _sc_kernels():
    mesh = plsc.VectorSubcoreMesh(core_axis_name="c", subcore_axis_name="s")

    @functools.partial(
        pl.kernel,
        mesh=mesh,
        out_type=[
            jax.ShapeDtypeStruct((NR, D), jnp.float32),
            jax.ShapeDtypeStruct((NR, GW), jnp.float32),
        ],
        scratch_types=[
            pltpu.VMEM((NCH, CH), jnp.int32),
            pltpu.VMEM((CH, D), jnp.float32),
            pltpu.VMEM((CH, GW), jnp.float32),
            pltpu.SemaphoreType.DMA,
            pltpu.SemaphoreType.DMA,
        ],
    )
    def scatter_k(slot_hbm, x_hbm, aux_hbm, buf_hbm, grow_hbm,
                  idx_v, row_v, aux_v, sem1, sem2):
        wid = lax.axis_index("s") * 2 + lax.axis_index("c")
        base = wid * TPW
        for j in range(NCH):
            pltpu.sync_copy(slot_hbm.at[pl.ds(base + j * CH, CH)], idx_v.at[j])
        for j in range(NCH):
            pltpu.sync_copy(x_hbm.at[pl.ds(base + j * CH, CH)], row_v)
            pltpu.async_copy(row_v, buf_hbm.at[idx_v.at[j]], sem1).wait()
            pltpu.sync_copy(aux_hbm.at[pl.ds(base + j * CH, CH)], aux_v)
            pltpu.async_copy(aux_v, grow_hbm.at[idx_v.at[j]], sem2).wait()

    @functools.partial(
        pl.kernel,
        mesh=mesh,
        out_type=jax.ShapeDtypeStruct((T, D), jnp.float32),
        scratch_types=[
            pltpu.VMEM((NCH, CH), jnp.int32),
            pltpu.VMEM((CH, D), jnp.float32),
            pltpu.SemaphoreType.DMA,
        ],
    )
    def gather_k(slot_hbm, y_hbm, out_hbm, idx_v, row_v, sem):
        wid = lax.axis_index("s") * 2 + lax.axis_index("c")
        base = wid * TPW
        for j in range(NCH):
            pltpu.sync_copy(slot_hbm.at[pl.ds(base + j * CH, CH)], idx_v.at[j])
        for j in range(NCH):
            pltpu.async_copy(y_hbm.at[idx_v.at[j]], row_v, sem).wait()
            pltpu.sync_copy(row_v, out_hbm.at[pl.ds(base + j * CH, CH)])

    return scatter_k, gather_k


def kernel(x, Wg, W1, b1, W2, b2):
    scatter_k, gather_k = _sc_kernels()
    slot2d, aux = _router(x, Wg)
    slot = slot2d.reshape(T)
    buf, grow = scatter_k(slot, x, aux)
    y = _ffn(buf, grow, W1, b1, W2, b2)
    return gather_k(slot, y)
